# baseline (speedup 1.0000x reference)
import functools

import jax
import jax.numpy as jnp
from jax import lax
from jax.experimental import pallas as pl
from jax.experimental.pallas import tpu as pltpu
from jax.experimental.pallas import tpu_sc as plsc

N = 10000
E = 320000
D = 128

NC = 2
NS = 16
CHUNK = 128
CT = 160
QT = CT // 4
E_PAD = NS * CT * CHUNK
N_PAD = 10240
RPT = N_PAD // NS
ZB = RPT // CHUNK
SUM_CORE = 0

_MESH = dict(core_axis_name="c", subcore_axis_name="s")


def _sc_kernel(x, src3, dst3):
    @functools.partial(
        pl.kernel,
        mesh=plsc.VectorSubcoreMesh(**_MESH),
        out_type=(
            jax.ShapeDtypeStruct((N_PAD, D), jnp.float32),
            jax.ShapeDtypeStruct((N_PAD, D), jnp.float32),
        ),
        scratch_types=[
            pltpu.VMEM((QT, CHUNK), jnp.int32),
            pltpu.VMEM((QT, CHUNK), jnp.int32),
            pltpu.VMEM((CHUNK, D), jnp.float32),
            pltpu.VMEM((CHUNK, D), jnp.float32),
            pltpu.VMEM_SHARED((N_PAD, D), jnp.float32),
            pltpu.SemaphoreType.DMA,
            pltpu.SemaphoreType.DMA,
        ],
    )
    def k(x_hbm, src_hbm, dst_hbm, psum_hbm, pcnt_hbm,
          idx_s, idx_d, rows0, rows1, acc_sh, sem0, sem1):
        c = lax.axis_index("c")
        s = lax.axis_index("s")
        z16 = jnp.zeros((16,), jnp.float32)
        o16 = jnp.ones((16,), jnp.float32)

        def fill_zero(r, carry):
            for jj in range(D // 16):
                rows0[r, pl.ds(jj * 16, 16)] = z16
            return carry

        lax.fori_loop(0, CHUNK, fill_zero, 0)
        for z in range(ZB):
            pltpu.sync_copy(rows0, acc_sh.at[pl.ds((s * ZB + z) * CHUNK, CHUNK)])
        plsc.subcore_barrier()

        @pl.when(c == SUM_CORE)
        def _sums():
            for q in range(4):
                pltpu.sync_copy(src_hbm.at[s, pl.ds(q * QT, QT)], idx_s)
                pltpu.sync_copy(dst_hbm.at[s, pl.ds(q * QT, QT)], idx_d)

                def body(g, carry):
                    pltpu.async_copy(x_hbm.at[idx_s.at[g]], rows0, sem0).wait()
                    pltpu.sync_copy(rows0, acc_sh.at[idx_d.at[g]], add=True)
                    return carry

                lax.fori_loop(0, QT, body, 0)

        @pl.when(c != SUM_CORE)
        def _counts():
            def fill_ones(r, carry):
                for jj in range(D // 16):
                    rows0[r, pl.ds(jj * 16, 16)] = o16
                return carry

            lax.fori_loop(0, CHUNK, fill_ones, 0)
            for q in range(4):
                pltpu.sync_copy(dst_hbm.at[s, pl.ds(q * QT, QT)], idx_d)

                def body(i, carry):
                    pltpu.sync_copy(rows0, acc_sh.at[idx_d.at[i]], add=True)
                    return carry

                lax.fori_loop(0, QT, body, 0)

        plsc.subcore_barrier()

        for z in range(ZB):
            r0 = (s * ZB + z) * CHUNK
            pltpu.sync_copy(acc_sh.at[pl.ds(r0, CHUNK)], rows1)

            @pl.when(c == SUM_CORE)
            def _():
                pltpu.sync_copy(rows1, psum_hbm.at[pl.ds(r0, CHUNK)])

            @pl.when(c != SUM_CORE)
            def _():
                pltpu.sync_copy(rows1, pcnt_hbm.at[pl.ds(r0, CHUNK)])

    return k(x, src3, dst3)


def _combine_kernel(psum, pcnt):
    BN = 2048

    def comb(ps_ref, pc_ref, o_ref):
        cnt = pc_ref[:, 0:1]
        o_ref[...] = ps_ref[...] / jnp.maximum(cnt, 1.0)

    return pl.pallas_call(
        comb,
        grid=(N_PAD // BN,),
        in_specs=[
            pl.BlockSpec((BN, D), lambda i: (i, 0)),
            pl.BlockSpec((BN, D), lambda i: (i, 0)),
        ],
        out_specs=pl.BlockSpec((BN, D), lambda i: (i, 0)),
        out_shape=jax.ShapeDtypeStruct((N_PAD, D), jnp.float32),
    )(psum, pcnt)


@jax.jit
def kernel(x, edge_index):
    src = edge_index[0]
    dst = edge_index[1]
    pad = E_PAD - E
    src_p = jnp.concatenate([src, jnp.zeros((pad,), jnp.int32)])
    dst_pad = N + (jnp.arange(pad, dtype=jnp.int32) % (N_PAD - N))
    dst_p = jnp.concatenate([dst, dst_pad])
    src3 = src_p.reshape(NS, CT, CHUNK)
    dst3 = dst_p.reshape(NS, CT, CHUNK)
    psum, pcnt = _sc_kernel(x, src3, dst3)
    return _combine_kernel(psum, pcnt)[:N]


# speedup vs baseline: 1.0662x; 1.0662x over previous
import functools

import jax
import jax.numpy as jnp
from jax import lax
from jax.experimental import pallas as pl
from jax.experimental.pallas import tpu as pltpu
from jax.experimental.pallas import tpu_sc as plsc

N = 10000
E = 320000
D = 128

NC = 2
NS = 16
NW = NC * NS
CHUNK = 128
CT = 80
HT = CT // 2
E_PAD = NW * CT * CHUNK
N_PAD = 10240
RPT = N_PAD // NS
ZB = RPT // CHUNK

_MESH = dict(core_axis_name="c", subcore_axis_name="s")


def _sc_sum_kernel(x, src3, dst3):
    @functools.partial(
        pl.kernel,
        mesh=plsc.VectorSubcoreMesh(**_MESH),
        out_type=jax.ShapeDtypeStruct((NC, N_PAD, D), jnp.float32),
        scratch_types=[
            pltpu.VMEM((HT, CHUNK), jnp.int32),
            pltpu.VMEM((HT, CHUNK), jnp.int32),
            pltpu.VMEM((CHUNK, D), jnp.float32),
            pltpu.VMEM_SHARED((N_PAD, D), jnp.float32),
            pltpu.SemaphoreType.DMA,
        ],
    )
    def k(x_hbm, src_hbm, dst_hbm, psum_hbm,
          idx_s, idx_d, rows_v, acc_sh, sem):
        c = lax.axis_index("c")
        s = lax.axis_index("s")
        wid = c * NS + s
        z16 = jnp.zeros((16,), jnp.float32)

        def fill_rows(r, carry):
            for jj in range(D // 16):
                rows_v[r, pl.ds(jj * 16, 16)] = z16
            return carry

        lax.fori_loop(0, CHUNK, fill_rows, 0)
        for z in range(ZB):
            pltpu.sync_copy(rows_v, acc_sh.at[pl.ds((s * ZB + z) * CHUNK, CHUNK)])
        plsc.subcore_barrier()

        for h in range(2):
            pltpu.sync_copy(src_hbm.at[wid, pl.ds(h * HT, HT)], idx_s)
            pltpu.sync_copy(dst_hbm.at[wid, pl.ds(h * HT, HT)], idx_d)

            def body(g, carry):
                pltpu.async_copy(x_hbm.at[idx_s.at[g]], rows_v, sem).wait()
                pltpu.sync_copy(rows_v, acc_sh.at[idx_d.at[g]], add=True)
                return carry

            lax.fori_loop(0, HT, body, 0)

        plsc.subcore_barrier()

        for z in range(ZB):
            r0 = (s * ZB + z) * CHUNK
            pltpu.sync_copy(acc_sh.at[pl.ds(r0, CHUNK)], rows_v)
            pltpu.sync_copy(rows_v, psum_hbm.at[c, pl.ds(r0, CHUNK)])

    return k(x, src3, dst3)


def _sc_count_kernel(dst3):
    @functools.partial(
        pl.kernel,
        mesh=plsc.VectorSubcoreMesh(**_MESH),
        out_type=jax.ShapeDtypeStruct((NC, N_PAD, D), jnp.float32),
        scratch_types=[
            pltpu.VMEM((HT, CHUNK), jnp.int32),
            pltpu.VMEM((CHUNK, D), jnp.float32),
            pltpu.VMEM((CHUNK, D), jnp.float32),
            pltpu.VMEM_SHARED((N_PAD, D), jnp.float32),
        ],
    )
    def k(dst_hbm, pcnt_hbm, idx_d, ones_v, buf_v, cnt_sh):
        c = lax.axis_index("c")
        s = lax.axis_index("s")
        wid = c * NS + s
        z16 = jnp.zeros((16,), jnp.float32)
        o16 = jnp.ones((16,), jnp.float32)

        def fill(r, carry):
            for jj in range(D // 16):
                ones_v[r, pl.ds(jj * 16, 16)] = o16
                buf_v[r, pl.ds(jj * 16, 16)] = z16
            return carry

        lax.fori_loop(0, CHUNK, fill, 0)
        for z in range(ZB):
            pltpu.sync_copy(buf_v, cnt_sh.at[pl.ds((s * ZB + z) * CHUNK, CHUNK)])
        plsc.subcore_barrier()

        for h in range(2):
            pltpu.sync_copy(dst_hbm.at[wid, pl.ds(h * HT, HT)], idx_d)

            def body(i, carry):
                pltpu.sync_copy(ones_v, cnt_sh.at[idx_d.at[i]], add=True)
                return carry

            lax.fori_loop(0, HT, body, 0)

        plsc.subcore_barrier()

        for z in range(ZB):
            r0 = (s * ZB + z) * CHUNK
            pltpu.sync_copy(cnt_sh.at[pl.ds(r0, CHUNK)], buf_v)
            pltpu.sync_copy(buf_v, pcnt_hbm.at[c, pl.ds(r0, CHUNK)])

    return k(dst3)


def _combine_kernel(psum, pcnt):
    BN = 2048

    def comb(ps_ref, pc_ref, o_ref):
        ssum = ps_ref[0] + ps_ref[1]
        cnt = pc_ref[0, :, 0:1] + pc_ref[1, :, 0:1]
        o_ref[...] = ssum / jnp.maximum(cnt, 1.0)

    return pl.pallas_call(
        comb,
        grid=(N_PAD // BN,),
        in_specs=[
            pl.BlockSpec((NC, BN, D), lambda i: (0, i, 0)),
            pl.BlockSpec((NC, BN, D), lambda i: (0, i, 0)),
        ],
        out_specs=pl.BlockSpec((BN, D), lambda i: (i, 0)),
        out_shape=jax.ShapeDtypeStruct((N_PAD, D), jnp.float32),
    )(psum, pcnt)


@jax.jit
def kernel(x, edge_index):
    src = edge_index[0]
    dst = edge_index[1]
    pad = E_PAD - E
    src_p = jnp.concatenate([src, jnp.zeros((pad,), jnp.int32)])
    dst_pad = N + (jnp.arange(pad, dtype=jnp.int32) % (N_PAD - N))
    dst_p = jnp.concatenate([dst, dst_pad])
    src3 = src_p.reshape(NW, CT, CHUNK)
    dst3 = dst_p.reshape(NW, CT, CHUNK)
    psum = _sc_sum_kernel(x, src3, dst3)
    pcnt = _sc_count_kernel(dst3)
    return _combine_kernel(psum, pcnt)[:N]


# speedup vs baseline: 1.2780x; 1.1986x over previous
import functools

import jax
import jax.numpy as jnp
from jax import lax
from jax.experimental import pallas as pl
from jax.experimental.pallas import tpu as pltpu
from jax.experimental.pallas import tpu_sc as plsc

N = 10000
E = 320000
D = 128

NC = 2
NS = 16
NW = NC * NS
CHUNK = 128
N_PAD = 10240
RPT = N_PAD // NS
ZB = RPT // CHUNK

_MESH = dict(core_axis_name="c", subcore_axis_name="s")


def _sc_sum_kernel(x, src, dst):
    e_pad = src.shape[0]
    chunks_per_tile = e_pad // (NW * CHUNK)

    @functools.partial(
        pl.kernel,
        mesh=plsc.VectorSubcoreMesh(**_MESH),
        out_type=jax.ShapeDtypeStruct((NC, N_PAD, D), jnp.float32),
        scratch_types=[
            pltpu.VMEM((CHUNK,), jnp.int32),
            pltpu.VMEM((CHUNK,), jnp.int32),
            pltpu.VMEM((CHUNK, D), jnp.float32),
            pltpu.VMEM_SHARED((N_PAD, D), jnp.float32),
            pltpu.SemaphoreType.DMA,
        ],
    )
    def k(x_hbm, src_hbm, dst_hbm, psum_hbm,
          idx_s, idx_d, rows_v, acc_sh, sem):
        c = lax.axis_index("c")
        s = lax.axis_index("s")
        wid = c * NS + s
        z16 = jnp.zeros((16,), jnp.float32)

        def fill_rows(r, carry):
            for jj in range(D // 16):
                rows_v[r, pl.ds(jj * 16, 16)] = z16
            return carry

        lax.fori_loop(0, CHUNK, fill_rows, 0)
        for z in range(ZB):
            pltpu.sync_copy(rows_v, acc_sh.at[pl.ds((s * ZB + z) * CHUNK, CHUNK)])
        plsc.subcore_barrier()

        def body(i, carry):
            base = (wid * chunks_per_tile + i) * CHUNK
            pltpu.sync_copy(src_hbm.at[pl.ds(base, CHUNK)], idx_s)
            pltpu.async_copy(x_hbm.at[idx_s], rows_v, sem).wait()
            pltpu.sync_copy(dst_hbm.at[pl.ds(base, CHUNK)], idx_d)
            pltpu.sync_copy(rows_v, acc_sh.at[idx_d], add=True)
            return carry

        lax.fori_loop(0, chunks_per_tile, body, 0)
        plsc.subcore_barrier()

        for z in range(ZB):
            r0 = (s * ZB + z) * CHUNK
            pltpu.sync_copy(acc_sh.at[pl.ds(r0, CHUNK)], rows_v)
            pltpu.sync_copy(rows_v, psum_hbm.at[c, pl.ds(r0, CHUNK)])

    return k(x, src, dst)


def _sc_count_kernel(dst):
    e_pad = dst.shape[0]
    chunks_per_tile = e_pad // (NW * CHUNK)

    @functools.partial(
        pl.kernel,
        mesh=plsc.VectorSubcoreMesh(**_MESH),
        out_type=jax.ShapeDtypeStruct((NC, N_PAD, D), jnp.float32),
        scratch_types=[
            pltpu.VMEM((CHUNK,), jnp.int32),
            pltpu.VMEM((CHUNK, D), jnp.float32),
            pltpu.VMEM((CHUNK, D), jnp.float32),
            pltpu.VMEM_SHARED((N_PAD, D), jnp.float32),
        ],
    )
    def k(dst_hbm, pcnt_hbm, idx_d, ones_v, buf_v, cnt_sh):
        c = lax.axis_index("c")
        s = lax.axis_index("s")
        wid = c * NS + s
        z16 = jnp.zeros((16,), jnp.float32)
        o16 = jnp.ones((16,), jnp.float32)

        def fill(r, carry):
            for jj in range(D // 16):
                ones_v[r, pl.ds(jj * 16, 16)] = o16
                buf_v[r, pl.ds(jj * 16, 16)] = z16
            return carry

        lax.fori_loop(0, CHUNK, fill, 0)
        for z in range(ZB):
            pltpu.sync_copy(buf_v, cnt_sh.at[pl.ds((s * ZB + z) * CHUNK, CHUNK)])
        plsc.subcore_barrier()

        def body(i, carry):
            base = (wid * chunks_per_tile + i) * CHUNK
            pltpu.sync_copy(dst_hbm.at[pl.ds(base, CHUNK)], idx_d)
            pltpu.sync_copy(ones_v, cnt_sh.at[idx_d], add=True)
            return carry

        lax.fori_loop(0, chunks_per_tile, body, 0)
        plsc.subcore_barrier()

        for z in range(ZB):
            r0 = (s * ZB + z) * CHUNK
            pltpu.sync_copy(cnt_sh.at[pl.ds(r0, CHUNK)], buf_v)
            pltpu.sync_copy(buf_v, pcnt_hbm.at[c, pl.ds(r0, CHUNK)])

    return k(dst)


def _combine_kernel(psum, pcnt):
    BN = 2048

    def comb(ps_ref, pc_ref, o_ref):
        ssum = ps_ref[0] + ps_ref[1]
        cnt = pc_ref[0, :, 0:1] + pc_ref[1, :, 0:1]
        o_ref[...] = ssum / jnp.maximum(cnt, 1.0)

    return pl.pallas_call(
        comb,
        grid=(N_PAD // BN,),
        in_specs=[
            pl.BlockSpec((NC, BN, D), lambda i: (0, i, 0)),
            pl.BlockSpec((NC, BN, D), lambda i: (0, i, 0)),
        ],
        out_specs=pl.BlockSpec((BN, D), lambda i: (i, 0)),
        out_shape=jax.ShapeDtypeStruct((N_PAD, D), jnp.float32),
    )(psum, pcnt)


@jax.jit
def kernel(x, edge_index):
    src = edge_index[0]
    dst = edge_index[1]
    e_pad = ((E + NW * CHUNK - 1) // (NW * CHUNK)) * (NW * CHUNK)
    pad = e_pad - E
    src_p = jnp.concatenate([src, jnp.zeros((pad,), jnp.int32)])
    dst_pad = N + (jnp.arange(pad, dtype=jnp.int32) % (N_PAD - N))
    dst_p = jnp.concatenate([dst, dst_pad])
    psum = _sc_sum_kernel(x, src_p, dst_p)
    pcnt = _sc_count_kernel(dst_p)
    return _combine_kernel(psum, pcnt)[:N]
